# baseline (speedup 1.0000x reference)
import functools

import jax
import jax.numpy as jnp
from jax import lax
from jax.experimental import pallas as pl
from jax.experimental.pallas import tpu as pltpu
from jax.experimental.pallas import tpu_sc as plsc

_B, _H, _W = 32, 512, 512
_ROWS = _B * _H
_NLBL = _B * _H * _W
_SCALE = 1.0 / float(_NLBL)

_NW = 32
_ROWS_PER_W = _ROWS // _NW
_CR = 16
_STEPS = _ROWS_PER_W // _CR
_GROUPS = _W // 16

_mesh = plsc.VectorSubcoreMesh(core_axis_name="c", subcore_axis_name="s")


@functools.partial(
    pl.kernel,
    mesh=_mesh,
    out_type=jax.ShapeDtypeStruct((_NW, 16), jnp.float32),
    scratch_types=[
        pltpu.VMEM((_CR * 2 * _W,), jnp.float32),
        pltpu.VMEM((_CR * 2 * _W,), jnp.float32),
        pltpu.VMEM((_CR * _W,), jnp.float32),
        pltpu.VMEM((_CR * _W,), jnp.float32),
        pltpu.VMEM((_CR * _W,), jnp.float32),
        pltpu.VMEM((_CR * _W,), jnp.float32),
        pltpu.VMEM((16,), jnp.float32),
        pltpu.SemaphoreType.DMA,
        pltpu.SemaphoreType.DMA,
        pltpu.SemaphoreType.DMA,
        pltpu.SemaphoreType.DMA,
        pltpu.SemaphoreType.DMA,
        pltpu.SemaphoreType.DMA,
    ],
    compiler_params=pltpu.CompilerParams(needs_layout_passes=False),
)
def _sc_partial(pred_hbm, reg_hbm, aff_hbm, out_hbm,
                pv0, pv1, rv0, rv1, av0, av1, accv,
                sp0, sp1, sr0, sr1, sa0, sa1):
    wid = lax.axis_index("s") * 2 + lax.axis_index("c")
    pbase = wid * (_ROWS_PER_W * 2 * _W)
    lbase = wid * (_ROWS_PER_W * _W)
    lane2 = lax.iota(jnp.int32, 16) * 2
    _PC = _CR * 2 * _W
    _LC = _CR * _W
    _GC = _LC // 16
    _UNROLL = 16

    def start(c, pv, rv, av, sp, sr, sa):
        pltpu.async_copy(pred_hbm.at[pl.ds(pbase + c * _PC, _PC)], pv, sp)
        pltpu.async_copy(reg_hbm.at[pl.ds(lbase + c * _LC, _LC)], rv, sr)
        pltpu.async_copy(aff_hbm.at[pl.ds(lbase + c * _LC, _LC)], av, sa)

    def wait(c, pv, rv, av, sp, sr, sa):
        pltpu.make_async_copy(pred_hbm.at[pl.ds(pbase + c * _PC, _PC)], pv, sp).wait()
        pltpu.make_async_copy(reg_hbm.at[pl.ds(lbase + c * _LC, _LC)], rv, sr).wait()
        pltpu.make_async_copy(aff_hbm.at[pl.ds(lbase + c * _LC, _LC)], av, sa).wait()

    def compute(pv, rv, av, acc):
        def blk(b, acc_b):
            accs = list(acc_b)
            for j in range(_UNROLL):
                col = lane2 + (b * (32 * _UNROLL) + 32 * j)
                p0 = plsc.load_gather(pv, [col])
                p1 = plsc.load_gather(pv, [col + 1])
                rr = rv[pl.ds(b * (16 * _UNROLL) + j * 16, 16)]
                aa = av[pl.ds(b * (16 * _UNROLL) + j * 16, 16)]
                d0 = p0 - rr
                d1 = p1 - aa
                accs[j % 8] = accs[j % 8] + d0 * d0 + d1 * d1
            return tuple(accs)

        return lax.fori_loop(0, _GC // _UNROLL, blk, acc)

    buf0 = (pv0, rv0, av0, sp0, sr0, sa0)
    buf1 = (pv1, rv1, av1, sp1, sr1, sa1)
    start(0, *buf0)
    start(1, *buf1)

    def pair_body(j, acc):
        wait(2 * j, *buf0)
        acc = compute(pv0, rv0, av0, acc)

        @pl.when(j < _STEPS // 2 - 1)
        def _():
            start(2 * j + 2, *buf0)

        wait(2 * j + 1, *buf1)
        acc = compute(pv1, rv1, av1, acc)

        @pl.when(j < _STEPS // 2 - 1)
        def _():
            start(2 * j + 3, *buf1)

        return acc

    zero = jnp.zeros(16, jnp.float32)
    acc = lax.fori_loop(0, _STEPS // 2, pair_body, (zero,) * 8)
    a = [acc[i] + acc[i + 4] for i in range(4)]
    accv[...] = (a[0] + a[1]) + (a[2] + a[3])
    pltpu.sync_copy(accv, out_hbm.at[wid])


def kernel(predicts, region_label, affinity_label):
    pred1d = predicts.reshape(-1)
    reg1d = region_label.reshape(-1)
    aff1d = affinity_label.reshape(-1)
    parts = _sc_partial(pred1d, reg1d, aff1d)
    return jnp.sum(parts) * jnp.float32(_SCALE)


# speedup vs baseline: 45.6439x; 45.6439x over previous
import functools

import jax
import jax.numpy as jnp
from jax import lax
from jax.experimental import pallas as pl
from jax.experimental.pallas import tpu as pltpu
from jax.experimental.pallas import tpu_sc as plsc

_B, _H, _W = 32, 512, 512
_ROWS = _B * _H
_NLBL = _B * _H * _W
_SCALE = 1.0 / float(_NLBL)

_NW = 32
_ROWS_PER_W = _ROWS // _NW
_CR = 16
_STEPS = _ROWS_PER_W // _CR
_GROUPS = _W // 16

_mesh = plsc.VectorSubcoreMesh(core_axis_name="c", subcore_axis_name="s")


@functools.partial(
    pl.kernel,
    mesh=_mesh,
    out_type=jax.ShapeDtypeStruct((_NW, 16), jnp.float32),
    scratch_types=[
        pltpu.VMEM((_CR, 2 * _W), jnp.float32),
        pltpu.VMEM((_CR, 2 * _W), jnp.float32),
        pltpu.VMEM((_CR, _W), jnp.float32),
        pltpu.VMEM((_CR, _W), jnp.float32),
        pltpu.VMEM((_CR, _W), jnp.float32),
        pltpu.VMEM((_CR, _W), jnp.float32),
        pltpu.VMEM((16,), jnp.float32),
        pltpu.SemaphoreType.DMA,
        pltpu.SemaphoreType.DMA,
        pltpu.SemaphoreType.DMA,
        pltpu.SemaphoreType.DMA,
        pltpu.SemaphoreType.DMA,
        pltpu.SemaphoreType.DMA,
    ],
    compiler_params=pltpu.CompilerParams(needs_layout_passes=False),
)
def _sc_partial(pred_hbm, reg_hbm, aff_hbm, out_hbm,
                pv0, pv1, rv0, rv1, av0, av1, accv,
                sp0, sp1, sr0, sr1, sa0, sa1):
    wid = lax.axis_index("s") * 2 + lax.axis_index("c")
    rbase = wid * _ROWS_PER_W
    lane2 = lax.iota(jnp.int32, 16) * 2
    _UNROLL = 16

    def start(row0, pv, rv, av, sp, sr, sa):
        pltpu.async_copy(pred_hbm.at[pl.ds(row0, _CR), :], pv, sp)
        pltpu.async_copy(reg_hbm.at[pl.ds(row0, _CR), :], rv, sr)
        pltpu.async_copy(aff_hbm.at[pl.ds(row0, _CR), :], av, sa)

    def wait(row0, pv, rv, av, sp, sr, sa):
        pltpu.make_async_copy(pred_hbm.at[pl.ds(row0, _CR), :], pv, sp).wait()
        pltpu.make_async_copy(reg_hbm.at[pl.ds(row0, _CR), :], rv, sr).wait()
        pltpu.make_async_copy(aff_hbm.at[pl.ds(row0, _CR), :], av, sa).wait()

    def compute(pv, rv, av, acc):
        def row_body(r, acc_r):
            row_splat = jnp.full((16,), r, jnp.int32)

            def blk(b, acc_b):
                accs = list(acc_b)
                for j in range(_UNROLL):
                    col = lane2 + (b * (32 * _UNROLL) + 32 * j)
                    p0 = plsc.load_gather(pv, [row_splat, col])
                    p1 = plsc.load_gather(pv, [row_splat, col + 1])
                    rr = rv[r, pl.ds(b * (16 * _UNROLL) + j * 16, 16)]
                    aa = av[r, pl.ds(b * (16 * _UNROLL) + j * 16, 16)]
                    d0 = p0 - rr
                    d1 = p1 - aa
                    accs[j % 8] = accs[j % 8] + d0 * d0 + d1 * d1
                return tuple(accs)

            return lax.fori_loop(0, _GROUPS // _UNROLL, blk, acc_r)

        return lax.fori_loop(0, _CR, row_body, acc)

    buf0 = (pv0, rv0, av0, sp0, sr0, sa0)
    buf1 = (pv1, rv1, av1, sp1, sr1, sa1)
    start(rbase, *buf0)
    start(rbase + _CR, *buf1)

    def pair_body(j, acc):
        row0 = rbase + (2 * j) * _CR
        wait(row0, *buf0)
        acc = compute(pv0, rv0, av0, acc)

        @pl.when(j < _STEPS // 2 - 1)
        def _():
            start(row0 + 2 * _CR, *buf0)

        row1 = row0 + _CR
        wait(row1, *buf1)
        acc = compute(pv1, rv1, av1, acc)

        @pl.when(j < _STEPS // 2 - 1)
        def _():
            start(row1 + 2 * _CR, *buf1)

        return acc

    zero = jnp.zeros(16, jnp.float32)
    acc = lax.fori_loop(0, _STEPS // 2, pair_body, (zero,) * 8)
    a = [acc[i] + acc[i + 4] for i in range(4)]
    accv[...] = (a[0] + a[1]) + (a[2] + a[3])
    pltpu.sync_copy(accv, out_hbm.at[wid])


def kernel(predicts, region_label, affinity_label):
    pred2d = predicts.reshape(_ROWS, 2 * _W)
    reg2d = region_label.reshape(_ROWS, _W)
    aff2d = affinity_label.reshape(_ROWS, _W)
    parts = _sc_partial(pred2d, reg2d, aff2d)
    return jnp.sum(parts) * jnp.float32(_SCALE)


# speedup vs baseline: 46.1702x; 1.0115x over previous
import functools

import jax
import jax.numpy as jnp
from jax import lax
from jax.experimental import pallas as pl
from jax.experimental.pallas import tpu as pltpu
from jax.experimental.pallas import tpu_sc as plsc

_B, _H, _W = 32, 512, 512
_ROWS = _B * _H
_NLBL = _B * _H * _W
_SCALE = 1.0 / float(_NLBL)

_NW = 32
_ROWS_PER_W = _ROWS // _NW
_CR = 16
_STEPS = _ROWS_PER_W // _CR
_GROUPS = _W // 16

_mesh = plsc.VectorSubcoreMesh(core_axis_name="c", subcore_axis_name="s")


@functools.partial(
    pl.kernel,
    mesh=_mesh,
    out_type=jax.ShapeDtypeStruct((_NW, 16), jnp.float32),
    scratch_types=[
        pltpu.VMEM((_CR, 2 * _W), jnp.float32),
        pltpu.VMEM((_CR, 2 * _W), jnp.float32),
        pltpu.VMEM((_CR, _W), jnp.float32),
        pltpu.VMEM((_CR, _W), jnp.float32),
        pltpu.VMEM((_CR, _W), jnp.float32),
        pltpu.VMEM((_CR, _W), jnp.float32),
        pltpu.VMEM((16,), jnp.float32),
        pltpu.SemaphoreType.DMA,
        pltpu.SemaphoreType.DMA,
        pltpu.SemaphoreType.DMA,
        pltpu.SemaphoreType.DMA,
        pltpu.SemaphoreType.DMA,
        pltpu.SemaphoreType.DMA,
    ],
    compiler_params=pltpu.CompilerParams(needs_layout_passes=False),
)
def _sc_partial(pred_hbm, reg_hbm, aff_hbm, out_hbm,
                pv0, pv1, rv0, rv1, av0, av1, accv,
                sp0, sp1, sr0, sr1, sa0, sa1):
    wid = lax.axis_index("s") * 2 + lax.axis_index("c")
    rbase = wid * _ROWS_PER_W
    lane2 = lax.iota(jnp.int32, 16) * 2
    _UNROLL = 8

    def start(row0, pv, rv, av, sp, sr, sa):
        pltpu.async_copy(pred_hbm.at[pl.ds(row0, _CR), :], pv, sp)
        pltpu.async_copy(reg_hbm.at[pl.ds(row0, _CR), :], rv, sr)
        pltpu.async_copy(aff_hbm.at[pl.ds(row0, _CR), :], av, sa)

    def wait(row0, pv, rv, av, sp, sr, sa):
        pltpu.make_async_copy(pred_hbm.at[pl.ds(row0, _CR), :], pv, sp).wait()
        pltpu.make_async_copy(reg_hbm.at[pl.ds(row0, _CR), :], rv, sr).wait()
        pltpu.make_async_copy(aff_hbm.at[pl.ds(row0, _CR), :], av, sa).wait()

    def compute(pv, rv, av, acc):
        def row_body(r, acc_r):
            row_splat = jnp.full((16,), r, jnp.int32)

            def blk(b, acc_b):
                accs = list(acc_b)
                for j in range(_UNROLL):
                    col = lane2 + (b * (32 * _UNROLL) + 32 * j)
                    p0 = plsc.load_gather(pv, [row_splat, col])
                    p1 = plsc.load_gather(pv, [row_splat, col + 1])
                    rr = rv[r, pl.ds(b * (16 * _UNROLL) + j * 16, 16)]
                    aa = av[r, pl.ds(b * (16 * _UNROLL) + j * 16, 16)]
                    d0 = p0 - rr
                    d1 = p1 - aa
                    accs[j % 8] = accs[j % 8] + d0 * d0 + d1 * d1
                return tuple(accs)

            return lax.fori_loop(0, _GROUPS // _UNROLL, blk, acc_r)

        return lax.fori_loop(0, _CR, row_body, acc)

    buf0 = (pv0, rv0, av0, sp0, sr0, sa0)
    buf1 = (pv1, rv1, av1, sp1, sr1, sa1)
    start(rbase, *buf0)
    start(rbase + _CR, *buf1)

    def pair_body(j, acc):
        row0 = rbase + (2 * j) * _CR
        wait(row0, *buf0)
        acc = compute(pv0, rv0, av0, acc)

        @pl.when(j < _STEPS // 2 - 1)
        def _():
            start(row0 + 2 * _CR, *buf0)

        row1 = row0 + _CR
        wait(row1, *buf1)
        acc = compute(pv1, rv1, av1, acc)

        @pl.when(j < _STEPS // 2 - 1)
        def _():
            start(row1 + 2 * _CR, *buf1)

        return acc

    zero = jnp.zeros(16, jnp.float32)
    acc = lax.fori_loop(0, _STEPS // 2, pair_body, (zero,) * 8)
    a = [acc[i] + acc[i + 4] for i in range(4)]
    accv[...] = (a[0] + a[1]) + (a[2] + a[3])
    pltpu.sync_copy(accv, out_hbm.at[wid])


def kernel(predicts, region_label, affinity_label):
    pred2d = predicts.reshape(_ROWS, 2 * _W)
    reg2d = region_label.reshape(_ROWS, _W)
    aff2d = affinity_label.reshape(_ROWS, _W)
    parts = _sc_partial(pred2d, reg2d, aff2d)
    return jnp.sum(parts) * jnp.float32(_SCALE)
